# Initial kernel scaffold; baseline (speedup 1.0000x reference)
import functools

import jax
import jax.numpy as jnp
from jax import lax
from jax.experimental import pallas as pl
from jax.experimental.pallas import tpu as pltpu
from jax.experimental.pallas import tpu_sc as plsc

_N = 10000
_E = 320000
_D = 128
_ALPHA = 0.1
_EPS = 1e-5

_NC = 2
_NS = 16
_L = 16
_NW = _NC * _NS

_CH = 128
_NCHT = 80
_EPAD = _NW * _NCHT * _CH
_NPAD = 10240
_RPT = _NPAD // _NS


def _rsqrt16(d):
    i = plsc.bitcast(d, jnp.int32)
    i = jnp.int32(0x5F3759DF) - lax.shift_right_logical(i, 1)
    y = plsc.bitcast(i, jnp.float32)
    for _ in range(3):
        y = y * (1.5 - 0.5 * d * y * y)
    return y


def _sc_gcn_agg(x, row3, col3, ew3):
    mesh = plsc.VectorSubcoreMesh(core_axis_name="c", subcore_axis_name="s")

    @functools.partial(
        pl.kernel,
        out_type=(
            jax.ShapeDtypeStruct((_NC, _NPAD, _D), jnp.float32),
            jax.ShapeDtypeStruct((_NPAD,), jnp.float32),
        ),
        mesh=mesh,
        scratch_types=[
            pltpu.VMEM((_NC, _NCHT, _CH), jnp.int32),
            pltpu.VMEM((_NC, _NCHT, _CH), jnp.float32),
            pltpu.VMEM((_NCHT, _CH), jnp.int32),
            pltpu.VMEM((_NPAD,), jnp.float32),
            pltpu.VMEM((_RPT, _L), jnp.float32),
            pltpu.VMEM((_CH, _L), jnp.float32),
            pltpu.VMEM((_CH, _D), jnp.float32),
            pltpu.VMEM((_CH,), jnp.float32),
            pltpu.VMEM((40, _D), jnp.float32),
            pltpu.VMEM_SHARED((_NPAD, _L), jnp.float32),
            pltpu.VMEM_SHARED((_NPAD, _D), jnp.float32),
        ],
    )
    def k(x_hbm, row_hbm, col_hbm, ew_hbm, agg_out, dinv_out,
          colbuf, ewbuf, rowbuf, dinvv, cbuf, valbuf, msgbuf, normbuf, zbuf,
          degmat, aggsh):
        c = lax.axis_index("c")
        s = lax.axis_index("s")
        iota16 = lax.iota(jnp.int32, _L)
        zero16i = jnp.zeros((_L,), jnp.int32)
        z16 = jnp.zeros((_L,), jnp.float32)

        @pl.loop(0, _RPT)
        def _(r):
            cbuf[r, :] = z16

        @pl.loop(0, 40)
        def _(r):
            for g in range(_D // _L):
                zbuf[r, pl.ds(g * _L, _L)] = z16

        pltpu.sync_copy(cbuf, degmat.at[pl.ds(s * _RPT, _RPT)])
        for i in range(_RPT // 40):
            pltpu.sync_copy(zbuf, aggsh.at[pl.ds(s * _RPT + i * 40, 40)])

        @pl.loop(0, _CH)
        def _(e):
            valbuf[e, :] = z16

        plsc.subcore_barrier()

        for h in range(_NC):
            slab = h * _NS + s
            pltpu.sync_copy(col_hbm.at[slab], colbuf.at[h])
            pltpu.sync_copy(ew_hbm.at[slab], ewbuf.at[h])

            @pl.loop(0, _NCHT)
            def _(j, h=h):
                for g in range(_CH // _L):
                    ew16 = ewbuf[h, j, pl.ds(g * _L, _L)]
                    plsc.store_scatter(valbuf, [g * _L + iota16, zero16i], ew16)
                pltpu.sync_copy(valbuf, degmat.at[colbuf.at[h, j]], add=True)

        plsc.subcore_barrier()

        for seg in range(_NS):
            pltpu.sync_copy(degmat.at[pl.ds(seg * _RPT, _RPT)], cbuf)

            @pl.loop(0, _RPT // _L)
            def _(g, seg=seg):
                r16 = g * _L + iota16
                d16 = plsc.load_gather(cbuf, [r16, zero16i])
                dinvv[pl.ds(seg * _RPT + g * _L, _L)] = _rsqrt16(d16 + 1.0)

        aslab = c * _NS + s
        pltpu.sync_copy(row_hbm.at[aslab], rowbuf)

        @pl.loop(0, _NCHT)
        def _(j):
            pltpu.sync_copy(x_hbm.at[rowbuf.at[j]], msgbuf)
            for g in range(_CH // _L):
                sl = pl.ds(g * _L, _L)
                r16 = rowbuf[j, sl]
                c16 = colbuf[c, j, sl]
                ew16 = ewbuf[c, j, sl]
                dr = plsc.load_gather(dinvv, [r16])
                dc = plsc.load_gather(dinvv, [c16])
                normbuf[sl] = dr * ew16 * dc

            @pl.loop(0, _CH)
            def _(e):
                ne = normbuf[e]
                for g in range(_D // _L):
                    sl = pl.ds(g * _L, _L)
                    msgbuf[e, sl] = msgbuf[e, sl] * ne

            pltpu.sync_copy(msgbuf, aggsh.at[colbuf.at[c, j]], add=True)

        plsc.subcore_barrier()

        pltpu.sync_copy(aggsh.at[pl.ds(s * _RPT, _RPT)],
                        agg_out.at[c, pl.ds(s * _RPT, _RPT)])

        @pl.when(c == 0)
        def _():
            pltpu.sync_copy(dinvv.at[pl.ds(s * _RPT, _RPT)],
                            dinv_out.at[pl.ds(s * _RPT, _RPT)])

    return k(x, row3, col3, ew3)


def _tc_tail(agg_ref, x_ref, x0_ref, dinv_ref, w_ref, g_ref, b_ref, y_ref):
    dsq = dinv_ref[...] * dinv_ref[...]
    agg = agg_ref[0] + agg_ref[1] + x_ref[...] * dsq
    h = (1.0 - _ALPHA) * agg + _ALPHA * x0_ref[...]
    out = jnp.dot(h, w_ref[...], preferred_element_type=jnp.float32,
                  precision=lax.Precision.HIGHEST)
    out = jnp.maximum(out, 0.0)
    mean = jnp.sum(out, axis=0) / _N
    msq = jnp.sum(out * out, axis=0) / _N
    var = msq - mean * mean
    scale = g_ref[...] * lax.rsqrt(var + _EPS)[None, :]
    y_ref[...] = (out - mean[None, :]) * scale + b_ref[...]


def kernel(x, x_orig, edge_index, edge_weight, W, gamma, beta):
    row = edge_index[0]
    col = edge_index[1]
    pad = _EPAD - _E
    padidx = jnp.arange(pad, dtype=jnp.int32) % _N
    zpad = jnp.zeros((pad,), dtype=jnp.float32)
    row3 = jnp.concatenate([row, padidx]).reshape(_NW, _NCHT, _CH)
    col3 = jnp.concatenate([col, padidx]).reshape(_NW, _NCHT, _CH)
    ew3 = jnp.concatenate([edge_weight, zpad]).reshape(_NW, _NCHT, _CH)

    aggp, dinv = _sc_gcn_agg(x, row3, col3, ew3)

    x_pad = jnp.pad(x, ((0, _NPAD - _N), (0, 0)))
    x0_pad = jnp.pad(x_orig, ((0, _NPAD - _N), (0, 0)))
    y_full = pl.pallas_call(
        _tc_tail,
        out_shape=jax.ShapeDtypeStruct((_NPAD, _D), jnp.float32),
    )(aggp, x_pad, x0_pad, dinv[:, None], W, gamma[None, :], beta[None, :])

    y = y_full[:_N]
    return (y, x_orig, edge_index, edge_weight, x)


# speedup vs baseline: 17.4393x; 17.4393x over previous
import dataclasses
import functools

import jax
import jax.numpy as jnp
from jax import lax
from jax.experimental import pallas as pl
from jax.experimental.pallas import tpu as pltpu
from jax.experimental.pallas import tpu_sc as plsc

_N = 10000
_E = 320000
_D = 128
_ALPHA = 0.1
_EPS = 1e-5

_NC = 2
_NS = 16
_L = 16
_NW = _NC * _NS

_CH = 128
_CB = 8
_NBLK = 10
_NCHT = _CB * _NBLK
_EPAD = _NW * _NCHT * _CH
_NPAD = 10240
_RPT = _NPAD // _NS
_CSEG = 160


def _rsqrt16(d):
    i = plsc.bitcast(d, jnp.int32)
    i = jnp.int32(0x5F3759DF) - lax.shift_right_logical(i, 1)
    y = plsc.bitcast(i, jnp.float32)
    for _ in range(3):
        y = y * (1.5 - 0.5 * d * y * y)
    return y


def _sc_gcn_agg(x, row3, col3, ew3):
    mesh = plsc.VectorSubcoreMesh(core_axis_name="c", subcore_axis_name="s")
    cp = pltpu.CompilerParams()
    if "needs_layout_passes" in pltpu.CompilerParams.__dataclass_fields__:
        cp = dataclasses.replace(cp, needs_layout_passes=False)
    if "use_tc_tiling_on_sc" in pltpu.CompilerParams.__dataclass_fields__:
        cp = dataclasses.replace(cp, use_tc_tiling_on_sc=False)

    @functools.partial(
        pl.kernel,
        compiler_params=cp,
        out_type=(
            jax.ShapeDtypeStruct((_NC, _NPAD, _D), jnp.float32),
            jax.ShapeDtypeStruct((_NPAD,), jnp.float32),
        ),
        mesh=mesh,
        scratch_types=[
            pltpu.VMEM((_CB, _CH), jnp.int32),
            pltpu.VMEM((_CB, _CH), jnp.int32),
            pltpu.VMEM((_CB, _CH), jnp.float32),
            pltpu.VMEM((_NPAD,), jnp.float32),
            pltpu.VMEM((_CSEG, _L), jnp.float32),
            pltpu.VMEM((_CH, _L), jnp.float32),
            pltpu.VMEM((_CH, _D), jnp.float32),
            pltpu.VMEM((_CH,), jnp.float32),
            pltpu.VMEM_SHARED((_NPAD, _L), jnp.float32),
            pltpu.VMEM_SHARED((_NPAD, _D), jnp.float32),
        ],
    )
    def k(x_hbm, row_hbm, col_hbm, ew_hbm, agg_out, dinv_out,
          rowbuf, colbuf, ewbuf, dinvv, cbuf, valbuf, msgbuf, normbuf,
          degmat, aggsh):
        c = lax.axis_index("c")
        s = lax.axis_index("s")
        iota16 = lax.iota(jnp.int32, _L)
        zero16i = jnp.zeros((_L,), jnp.int32)
        z16 = jnp.zeros((_L,), jnp.float32)

        @pl.loop(0, _CH)
        def _(e):
            valbuf[e, :] = z16
            for g in range(_D // _L):
                msgbuf[e, pl.ds(g * _L, _L)] = z16

        for i in range(_RPT // _CH):
            pltpu.sync_copy(valbuf, degmat.at[pl.ds(s * _RPT + i * _CH, _CH)])
            pltpu.sync_copy(msgbuf, aggsh.at[pl.ds(s * _RPT + i * _CH, _CH)])

        plsc.subcore_barrier()

        for h in range(_NC):
            slab = h * _NS + s

            @pl.loop(0, _NBLK)
            def _(jb, slab=slab):
                pltpu.sync_copy(col_hbm.at[slab, pl.ds(jb * _CB, _CB)], colbuf)
                pltpu.sync_copy(ew_hbm.at[slab, pl.ds(jb * _CB, _CB)], ewbuf)
                for j8 in range(_CB):
                    for g in range(_CH // _L):
                        ew16 = ewbuf[j8, pl.ds(g * _L, _L)]
                        plsc.store_scatter(valbuf, [g * _L + iota16, zero16i],
                                           ew16)
                    pltpu.sync_copy(valbuf, degmat.at[colbuf.at[j8]], add=True)

        plsc.subcore_barrier()

        @pl.loop(0, _NPAD // _CSEG)
        def _(seg):
            pltpu.sync_copy(degmat.at[pl.ds(seg * _CSEG, _CSEG)], cbuf)

            @pl.loop(0, _CSEG // _L)
            def _(g, seg=seg):
                r16 = g * _L + iota16
                d16 = plsc.load_gather(cbuf, [r16, zero16i])
                dinvv[pl.ds(seg * _CSEG + g * _L, _L)] = _rsqrt16(d16 + 1.0)

        aslab = c * _NS + s

        @pl.loop(0, _NBLK)
        def _(jb):
            pltpu.sync_copy(row_hbm.at[aslab, pl.ds(jb * _CB, _CB)], rowbuf)
            pltpu.sync_copy(col_hbm.at[aslab, pl.ds(jb * _CB, _CB)], colbuf)
            pltpu.sync_copy(ew_hbm.at[aslab, pl.ds(jb * _CB, _CB)], ewbuf)
            for j8 in range(_CB):
                pltpu.sync_copy(x_hbm.at[rowbuf.at[j8]], msgbuf)
                for g in range(_CH // _L):
                    sl = pl.ds(g * _L, _L)
                    r16 = rowbuf[j8, sl]
                    c16 = colbuf[j8, sl]
                    ew16 = ewbuf[j8, sl]
                    dr = plsc.load_gather(dinvv, [r16])
                    dc = plsc.load_gather(dinvv, [c16])
                    normbuf[sl] = dr * ew16 * dc

                @pl.loop(0, _CH // _L)
                def _(eo, j8=j8):
                    n16 = normbuf[pl.ds(eo * _L, _L)]
                    for kk in range(_L):
                        ne = n16[kk]
                        for g in range(_D // _L):
                            sl = pl.ds(g * _L, _L)
                            msgbuf[eo * _L + kk, sl] = \
                                msgbuf[eo * _L + kk, sl] * ne

                pltpu.sync_copy(msgbuf, aggsh.at[colbuf.at[j8]], add=True)

        plsc.subcore_barrier()

        pltpu.sync_copy(aggsh.at[pl.ds(s * _RPT, _RPT)],
                        agg_out.at[c, pl.ds(s * _RPT, _RPT)])

        @pl.when(c == 0)
        def _():
            pltpu.sync_copy(dinvv.at[pl.ds(s * _RPT, _RPT)],
                            dinv_out.at[pl.ds(s * _RPT, _RPT)])

    return k(x, row3, col3, ew3)


def _tc_tail(agg_ref, x_ref, x0_ref, dinv_ref, w_ref, g_ref, b_ref, y_ref):
    dsq = dinv_ref[...] * dinv_ref[...]
    agg = agg_ref[0] + agg_ref[1] + x_ref[...] * dsq
    h = (1.0 - _ALPHA) * agg + _ALPHA * x0_ref[...]
    out = jnp.dot(h, w_ref[...], preferred_element_type=jnp.float32,
                  precision=lax.Precision.HIGHEST)
    out = jnp.maximum(out, 0.0)
    mean = jnp.sum(out, axis=0) / _N
    msq = jnp.sum(out * out, axis=0) / _N
    var = msq - mean * mean
    scale = g_ref[...] * lax.rsqrt(var + _EPS)[None, :]
    y_ref[...] = (out - mean[None, :]) * scale + b_ref[...]


def kernel(x, x_orig, edge_index, edge_weight, W, gamma, beta):
    row = edge_index[0]
    col = edge_index[1]
    pad = _EPAD - _E
    padidx = jnp.arange(pad, dtype=jnp.int32) % _N
    zpad = jnp.zeros((pad,), dtype=jnp.float32)
    row3 = jnp.concatenate([row, padidx]).reshape(_NW, _NCHT, _CH)
    col3 = jnp.concatenate([col, padidx]).reshape(_NW, _NCHT, _CH)
    ew3 = jnp.concatenate([edge_weight, zpad]).reshape(_NW, _NCHT, _CH)

    aggp, dinv = _sc_gcn_agg(x, row3, col3, ew3)

    x_pad = jnp.pad(x, ((0, _NPAD - _N), (0, 0)))
    x0_pad = jnp.pad(x_orig, ((0, _NPAD - _N), (0, 0)))
    y_full = pl.pallas_call(
        _tc_tail,
        out_shape=jax.ShapeDtypeStruct((_NPAD, _D), jnp.float32),
    )(aggp, x_pad, x0_pad, dinv[:, None], W, gamma[None, :], beta[None, :])

    y = y_full[:_N]
    return (y, x_orig, edge_index, edge_weight, x)
